# Initial kernel scaffold; baseline (speedup 1.0000x reference)
import jax
import jax.numpy as jnp
from jax.experimental import pallas as pl


def kernel(x, pos, idx, src0, dst0, src1, dst1, W, b):
    raise NotImplementedError("write your pallas kernel here")


# speedup vs baseline: 5.9603x; 5.9603x over previous
import functools

import jax
import jax.numpy as jnp
from jax import lax
from jax.experimental import pallas as pl
from jax.experimental.pallas import tpu as pltpu
from jax.experimental.pallas import tpu_sc as plsc

_NT = 32
_CHUNK = 128
_WSEG = 128
_CROWS = 160
_NEG_INF = float("-inf")


def _tc_uv(x, posp, w1, w2p, b2):
    n, d = x.shape
    blk = 2000

    def body(x_ref, p_ref, w1_ref, w2_ref, b_ref, u_ref, v_ref):
        pw = jnp.dot(p_ref[...], w2_ref[...], preferred_element_type=jnp.float32)
        u_ref[...] = jnp.dot(x_ref[...], w1_ref[...],
                             preferred_element_type=jnp.float32) + pw
        v_ref[...] = b_ref[...] - pw

    return pl.pallas_call(
        body,
        grid=(n // blk,),
        in_specs=[
            pl.BlockSpec((blk, d), lambda i: (i, 0)),
            pl.BlockSpec((blk, 128), lambda i: (i, 0)),
            pl.BlockSpec((d, 128), lambda i: (0, 0)),
            pl.BlockSpec((128, 128), lambda i: (0, 0)),
            pl.BlockSpec((1, 128), lambda i: (0, 0)),
        ],
        out_specs=[pl.BlockSpec((blk, 128), lambda i: (i, 0)),
                   pl.BlockSpec((blk, 128), lambda i: (i, 0))],
        out_shape=[jax.ShapeDtypeStruct((n, 128), jnp.float32)] * 2,
    )(x, posp, w1, w2p, b2)


def _tc_epilogue(raw, c):
    rows = raw.shape[0]
    blk = 512

    def body(r_ref, c_ref, o_ref):
        o_ref[...] = jnp.maximum(r_ref[...] + c_ref[...], 0.0)

    return pl.pallas_call(
        body,
        grid=(rows // blk,),
        in_specs=[pl.BlockSpec((blk, 128), lambda i: (i, 0)),
                  pl.BlockSpec((blk, 128), lambda i: (i, 0))],
        out_specs=pl.BlockSpec((blk, 128), lambda i: (i, 0)),
        out_shape=jax.ShapeDtypeStruct((rows, 128), jnp.float32),
    )(raw, c)


def _make_meta(dst_all, etot, nseg2):
    t = jnp.arange(_NT)
    eb = (t * etot) // _NT
    lo = dst_all[eb].at[0].set(0).astype(jnp.int32)
    hi = jnp.concatenate([lo[1:], jnp.array([nseg2], jnp.int32)])
    e_lo = jnp.searchsorted(dst_all, lo, side='left').astype(jnp.int32)
    e_hi = jnp.searchsorted(dst_all, hi, side='left').astype(jnp.int32)
    c0 = (e_lo // _CHUNK) * _CHUNK
    nch = jnp.where(e_hi > e_lo, (e_hi - c0 + _CHUNK - 1) // _CHUNK, 0)
    z = jnp.zeros_like(c0)
    return jnp.stack([c0, nch, e_lo, e_hi, lo, hi, z, z, z, z, z, z, z, z,
                      z, z], axis=1).astype(jnp.int32)


def _sc_segmax(u, v, idx2, src_all, dst_all, meta, nseg2, nrows):
    mesh = plsc.VectorSubcoreMesh(core_axis_name="c", subcore_axis_name="s",
                                  num_cores=2, num_subcores=16)

    @functools.partial(
        pl.kernel,
        out_type=[jax.ShapeDtypeStruct((nrows, 128), jnp.float32),
                  jax.ShapeDtypeStruct((nrows, 128), jnp.float32)],
        mesh=mesh,
        scratch_types=[
            pltpu.VMEM((16,), jnp.int32),
            pltpu.VMEM((_CHUNK,), jnp.int32),
            pltpu.VMEM((_CHUNK + 16,), jnp.int32),
            pltpu.VMEM((_CHUNK, 128), jnp.float32),
            pltpu.VMEM((_WSEG, 128), jnp.float32),
            pltpu.VMEM((_WSEG,), jnp.int32),
            pltpu.VMEM((_CROWS // 2,), jnp.int32),
            pltpu.VMEM((_CROWS // 2,), jnp.int32),
            pltpu.VMEM((_CROWS // 2, 128), jnp.float32),
            pltpu.SemaphoreType.DMA,
            pltpu.SemaphoreType.DMA,
        ],
    )
    def k(u_hbm, v_hbm, idx_hbm, src_hbm, dst_hbm, meta_hbm,
          raw_hbm, c_hbm,
          meta_v, src_v, dst_v, rows_v, win_v, sidx_v,
          cgi_v, civ_v, crow_v, gsem, ssem):
        wid = lax.axis_index("s") * 2 + lax.axis_index("c")
        iota = lax.iota(jnp.int32, 16)
        ninf_vec = jnp.full((16,), _NEG_INF, jnp.float32)
        half = _CROWS // 2

        cbase = pl.multiple_of(wid * _CROWS, 8)
        for h in range(2):
            for j in range(half // 16):
                g = cbase + h * half + j * 16 + iota
                cgi_v[pl.ds(j * 16, 16)] = jnp.minimum(g, nseg2 - 1)
            pltpu.async_copy(idx_hbm.at[cgi_v], civ_v, gsem).wait()
            pltpu.async_copy(v_hbm.at[civ_v], crow_v, gsem).wait()
            pltpu.sync_copy(crow_v,
                            c_hbm.at[pl.ds(cbase + h * half, half)])

        pltpu.sync_copy(meta_hbm.at[wid], meta_v)
        mvec = meta_v[...]
        c0 = mvec[0]
        nch = mvec[1]
        e_lo = mvec[2]
        e_hi = mvec[3]
        s_lo = mvec[4]
        s_hi = mvec[5]

        def _reinit_win():
            @pl.loop(0, _WSEG)
            def _ib(r):
                wr = win_v.at[r]
                for j in range(8):
                    wr[pl.ds(j * 16, 16)] = ninf_vec

        _reinit_win()

        def flush_window(wbase):
            for j in range(8):
                g = wbase + j * 16 + iota
                sidx_v[pl.ds(j * 16, 16)] = jnp.where(g < s_hi, g, nseg2)
            pltpu.async_copy(win_v, raw_hbm.at[sidx_v], ssem).wait()
            _reinit_win()
            return wbase + _WSEG

        @pl.loop(0, nch, init_carry=(s_lo, s_lo) + tuple(ninf_vec
                                                         for _ in range(8)))
        def chunk_loop(ci, carry):
            cur, wbase = carry[0], carry[1]
            acc = list(carry[2:])
            cstart = pl.multiple_of(c0 + ci * _CHUNK, _CHUNK)
            pltpu.sync_copy(src_hbm.at[pl.ds(cstart, _CHUNK)], src_v)
            pltpu.sync_copy(dst_hbm.at[pl.ds(cstart, _CHUNK)],
                            dst_v.at[pl.ds(0, _CHUNK)])
            pltpu.async_copy(u_hbm.at[src_v], rows_v, gsem).wait()
            lo = jnp.maximum(e_lo, cstart)
            hi = jnp.minimum(e_hi, cstart + _CHUNK)

            @pl.loop(lo, hi, init_carry=(cur, wbase, *acc))
            def edge_loop(e, ec):
                cur, wbase = ec[0], ec[1]
                acc = list(ec[2:])
                o = e - cstart
                d = dst_v[pl.ds(o, 16)][0]
                ro = rows_v.at[o]
                rv = [ro[pl.ds(j * 16, 16)] for j in range(8)]
                same = d == cur

                @pl.when(jnp.logical_not(same))
                def _():
                    wr = win_v.at[cur - wbase]
                    for jj in range(8):
                        wr[pl.ds(jj * 16, 16)] = acc[jj]

                nadv = (d - wbase) // _WSEG

                @pl.loop(0, nadv, init_carry=wbase)
                def wadv(i, wb):
                    return flush_window(wb)
                new_wbase = jnp.where(nadv > 0, wadv, wbase)

                new_acc = [
                    jnp.maximum(jnp.where(same, acc[j], ninf_vec), rv[j])
                    for j in range(8)
                ]
                return (d, new_wbase, *new_acc)

            return edge_loop

        cur, wbase = chunk_loop[0], chunk_loop[1]
        acc = list(chunk_loop[2:])

        @pl.when(e_hi > e_lo)
        def _():
            wr = win_v.at[cur - wbase]
            for j in range(8):
                wr[pl.ds(j * 16, 16)] = acc[j]

        nrem = (s_hi - wbase + _WSEG - 1) // _WSEG

        @pl.loop(0, nrem, init_carry=wbase)
        def wfin(i, wb):
            return flush_window(wb)

    return k(u, v, idx2, src_all, dst_all, meta)


@jax.jit
def kernel(x, pos, idx, src0, dst0, src1, dst1, W, b):
    n, d = x.shape
    nseg = idx.shape[0]
    posp = jnp.zeros((n, 128), jnp.float32).at[:, :3].set(pos)
    w1 = W[:d]
    w2p = jnp.zeros((128, 128), jnp.float32).at[:3].set(W[d:])
    b2 = b.reshape(1, 128)
    u, v = _tc_uv(x, posp, w1, w2p, b2)

    idx2 = jnp.concatenate([idx, idx]).astype(jnp.int32)
    src_all = jnp.concatenate([src0, src1]).astype(jnp.int32)
    dst_all = jnp.concatenate(
        [dst0.astype(jnp.int32), dst1.astype(jnp.int32) + nseg])
    nseg2 = 2 * nseg
    nrows = _NT * _CROWS
    meta = _make_meta(dst_all, src_all.shape[0], nseg2)

    raw, c = _sc_segmax(u, v, idx2, src_all, dst_all, meta, nseg2, nrows)
    out = _tc_epilogue(raw, c)
    return jnp.concatenate([out[:nseg], out[nseg:nseg2]], axis=-1)


# speedup vs baseline: 7.7991x; 1.3085x over previous
import functools

import jax
import jax.numpy as jnp
from jax import lax
from jax.experimental import pallas as pl
from jax.experimental.pallas import tpu as pltpu
from jax.experimental.pallas import tpu_sc as plsc

_NT = 32
_CHUNK = 128
_WSEG = 128
_CROWS = 160
_NEG_INF = float("-inf")


def _tc_uv(x, posp, w1, w2p, b2):
    n, d = x.shape
    blk = 2000

    def body(x_ref, p_ref, w1_ref, w2_ref, b_ref, u_ref, v_ref):
        pw = jnp.dot(p_ref[...], w2_ref[...], preferred_element_type=jnp.float32)
        u_ref[...] = jnp.dot(x_ref[...], w1_ref[...],
                             preferred_element_type=jnp.float32) + pw
        v_ref[...] = b_ref[...] - pw

    return pl.pallas_call(
        body,
        grid=(n // blk,),
        in_specs=[
            pl.BlockSpec((blk, d), lambda i: (i, 0)),
            pl.BlockSpec((blk, 128), lambda i: (i, 0)),
            pl.BlockSpec((d, 128), lambda i: (0, 0)),
            pl.BlockSpec((128, 128), lambda i: (0, 0)),
            pl.BlockSpec((1, 128), lambda i: (0, 0)),
        ],
        out_specs=[pl.BlockSpec((blk, 128), lambda i: (i, 0)),
                   pl.BlockSpec((blk, 128), lambda i: (i, 0))],
        out_shape=[jax.ShapeDtypeStruct((n, 128), jnp.float32)] * 2,
    )(x, posp, w1, w2p, b2)


def _tc_epilogue(raw, c):
    rows = raw.shape[0]
    blk = 512

    def body(r_ref, c_ref, o_ref):
        o_ref[...] = jnp.maximum(r_ref[...] + c_ref[...], 0.0)

    return pl.pallas_call(
        body,
        grid=(rows // blk,),
        in_specs=[pl.BlockSpec((blk, 128), lambda i: (i, 0)),
                  pl.BlockSpec((blk, 128), lambda i: (i, 0))],
        out_specs=pl.BlockSpec((blk, 128), lambda i: (i, 0)),
        out_shape=jax.ShapeDtypeStruct((rows, 128), jnp.float32),
    )(raw, c)


def _make_meta(dst_all, etot, nseg2):
    t = jnp.arange(_NT)
    eb = (t * etot) // _NT
    lo = dst_all[eb].at[0].set(0).astype(jnp.int32)
    hi = jnp.concatenate([lo[1:], jnp.array([nseg2], jnp.int32)])
    e_lo = jnp.searchsorted(dst_all, lo, side='left').astype(jnp.int32)
    e_hi = jnp.searchsorted(dst_all, hi, side='left').astype(jnp.int32)
    c0 = (e_lo // _CHUNK) * _CHUNK
    nch = jnp.where(e_hi > e_lo, (e_hi - c0 + _CHUNK - 1) // _CHUNK, 0)
    z = jnp.zeros_like(c0)
    return jnp.stack([c0, nch, e_lo, e_hi, lo, hi, z, z, z, z, z, z, z, z,
                      z, z], axis=1).astype(jnp.int32)


def _sc_segmax(u, v, idx2, src_all, dst_all, meta, nseg2, nrows):
    mesh = plsc.VectorSubcoreMesh(core_axis_name="c", subcore_axis_name="s",
                                  num_cores=2, num_subcores=16)

    @functools.partial(
        pl.kernel,
        out_type=[jax.ShapeDtypeStruct((nrows, 128), jnp.float32),
                  jax.ShapeDtypeStruct((nrows, 128), jnp.float32)],
        mesh=mesh,
        scratch_types=[
            pltpu.VMEM((16,), jnp.int32),
            pltpu.VMEM((_CHUNK,), jnp.int32),
            pltpu.VMEM((_CHUNK,), jnp.int32),
            pltpu.VMEM((_CHUNK + 16,), jnp.int32),
            pltpu.VMEM((_CHUNK + 16,), jnp.int32),
            pltpu.VMEM((_CHUNK, 128), jnp.float32),
            pltpu.VMEM((_CHUNK, 128), jnp.float32),
            pltpu.VMEM((_WSEG, 128), jnp.float32),
            pltpu.VMEM((_WSEG,), jnp.int32),
            pltpu.VMEM((_CROWS // 2,), jnp.int32),
            pltpu.VMEM((_CROWS // 2,), jnp.int32),
            pltpu.VMEM((_CROWS // 2, 128), jnp.float32),
            pltpu.SemaphoreType.DMA,
            pltpu.SemaphoreType.DMA,
            pltpu.SemaphoreType.DMA,
            pltpu.SemaphoreType.DMA,
        ],
    )
    def k(u_hbm, v_hbm, idx_hbm, src_hbm, dst_hbm, meta_hbm,
          raw_hbm, c_hbm,
          meta_v, src_a, src_b, dst_a, dst_b, rows_a, rows_b,
          win_v, sidx_v, cgi_v, civ_v, crow_v, gsem, isem_a, isem_b, ssem):
        wid = lax.axis_index("s") * 2 + lax.axis_index("c")
        iota = lax.iota(jnp.int32, 16)
        ninf_vec = jnp.full((16,), _NEG_INF, jnp.float32)
        half = _CROWS // 2

        cbase = pl.multiple_of(wid * _CROWS, 8)
        for h in range(2):
            for j in range(half // 16):
                g = cbase + h * half + j * 16 + iota
                cgi_v[pl.ds(j * 16, 16)] = jnp.minimum(g, nseg2 - 1)
            pltpu.async_copy(idx_hbm.at[cgi_v], civ_v, gsem).wait()
            pltpu.async_copy(v_hbm.at[civ_v], crow_v, gsem).wait()
            pltpu.sync_copy(crow_v,
                            c_hbm.at[pl.ds(cbase + h * half, half)])

        pltpu.sync_copy(meta_hbm.at[wid], meta_v)
        mvec = meta_v[...]
        c0 = mvec[0]
        nch = mvec[1]
        e_lo = mvec[2]
        e_hi = mvec[3]
        s_lo = mvec[4]
        s_hi = mvec[5]
        last = jnp.maximum(nch - 1, 0)

        def cstart_of(ci):
            return pl.multiple_of(c0 + jnp.minimum(ci, last) * _CHUNK, _CHUNK)

        def issue_sd(ci, src_v, dst_v, isem):
            cs = cstart_of(ci)
            pltpu.async_copy(src_hbm.at[pl.ds(cs, _CHUNK)], src_v, isem)
            pltpu.async_copy(dst_hbm.at[pl.ds(cs, _CHUNK)],
                             dst_v.at[pl.ds(0, _CHUNK)], isem)

        def wait_sd(src_v, dst_v, isem):
            pltpu.make_async_copy(src_hbm.at[pl.ds(0, _CHUNK)], src_v,
                                  isem).wait()
            pltpu.make_async_copy(dst_hbm.at[pl.ds(0, _CHUNK)],
                                  dst_v.at[pl.ds(0, _CHUNK)], isem).wait()

        def issue_g(src_v, rows_v):
            pltpu.async_copy(u_hbm.at[src_v], rows_v, gsem)

        def wait_g(src_v, rows_v):
            pltpu.make_async_copy(u_hbm.at[src_v], rows_v, gsem).wait()

        def _reinit_win():
            @pl.loop(0, _WSEG)
            def _ib(r):
                wr = win_v.at[r]
                for j in range(8):
                    wr[pl.ds(j * 16, 16)] = ninf_vec

        _reinit_win()

        def flush_window(wbase):
            for j in range(8):
                g = wbase + j * 16 + iota
                sidx_v[pl.ds(j * 16, 16)] = jnp.where(g < s_hi, g, nseg2)
            pltpu.async_copy(win_v, raw_hbm.at[sidx_v], ssem).wait()
            _reinit_win()
            return wbase + _WSEG

        def scan_chunk(ci, carry, src_v, dst_v, rows_v):
            cur, wbase = carry[0], carry[1]
            acc = list(carry[2:])
            cstart = cstart_of(ci)
            lo = jnp.maximum(e_lo, cstart)
            hi = jnp.where(ci < nch, jnp.minimum(e_hi, cstart + _CHUNK), lo)

            @pl.loop(lo, hi, init_carry=(cur, wbase, *acc))
            def edge_loop(e, ec):
                cur, wbase = ec[0], ec[1]
                acc = list(ec[2:])
                o = e - cstart
                d = dst_v[pl.ds(o, 16)][0]
                ro = rows_v.at[o]
                rv = [ro[pl.ds(j * 16, 16)] for j in range(8)]
                same = d == cur

                @pl.when(jnp.logical_not(same))
                def _():
                    wr = win_v.at[cur - wbase]
                    for jj in range(8):
                        wr[pl.ds(jj * 16, 16)] = acc[jj]

                nadv = (d - wbase) // _WSEG

                @pl.loop(0, nadv, init_carry=wbase)
                def wadv(i, wb):
                    return flush_window(wb)

                new_acc = [
                    jnp.maximum(jnp.where(same, acc[j], ninf_vec), rv[j])
                    for j in range(8)
                ]
                return (d, wadv, *new_acc)

            return edge_loop

        npairs = (nch + 1) // 2

        issue_sd(0, src_a, dst_a, isem_a)
        wait_sd(src_a, dst_a, isem_a)
        issue_g(src_a, rows_a)
        issue_sd(1, src_b, dst_b, isem_b)

        init = (s_lo, s_lo) + tuple(ninf_vec for _ in range(8))

        @pl.loop(0, npairs, init_carry=init)
        def pair_loop(t, carry):
            ca = 2 * t
            wait_g(src_a, rows_a)
            wait_sd(src_b, dst_b, isem_b)
            issue_g(src_b, rows_b)
            carry = scan_chunk(ca, carry, src_a, dst_a, rows_a)
            issue_sd(ca + 2, src_a, dst_a, isem_a)
            wait_g(src_b, rows_b)
            wait_sd(src_a, dst_a, isem_a)
            issue_g(src_a, rows_a)
            carry = scan_chunk(ca + 1, carry, src_b, dst_b, rows_b)
            issue_sd(ca + 3, src_b, dst_b, isem_b)
            return carry

        wait_g(src_a, rows_a)
        wait_sd(src_b, dst_b, isem_b)

        cur, wbase = pair_loop[0], pair_loop[1]
        acc = list(pair_loop[2:])

        @pl.when(e_hi > e_lo)
        def _():
            wr = win_v.at[cur - wbase]
            for j in range(8):
                wr[pl.ds(j * 16, 16)] = acc[j]

        nrem = (s_hi - wbase + _WSEG - 1) // _WSEG

        @pl.loop(0, nrem, init_carry=wbase)
        def wfin(i, wb):
            return flush_window(wb)

    return k(u, v, idx2, src_all, dst_all, meta)


@jax.jit
def kernel(x, pos, idx, src0, dst0, src1, dst1, W, b):
    n, d = x.shape
    nseg = idx.shape[0]
    posp = jnp.zeros((n, 128), jnp.float32).at[:, :3].set(pos)
    w1 = W[:d]
    w2p = jnp.zeros((128, 128), jnp.float32).at[:3].set(W[d:])
    b2 = b.reshape(1, 128)
    u, v = _tc_uv(x, posp, w1, w2p, b2)

    idx2 = jnp.concatenate([idx, idx]).astype(jnp.int32)
    src_all = jnp.concatenate([src0, src1]).astype(jnp.int32)
    dst_all = jnp.concatenate(
        [dst0.astype(jnp.int32), dst1.astype(jnp.int32) + nseg])
    nseg2 = 2 * nseg
    nrows = _NT * _CROWS
    meta = _make_meta(dst_all, src_all.shape[0], nseg2)

    raw, c = _sc_segmax(u, v, idx2, src_all, dst_all, meta, nseg2, nrows)
    out = _tc_epilogue(raw, c)
    return jnp.concatenate([out[:nseg], out[nseg:nseg2]], axis=-1)


# speedup vs baseline: 8.2340x; 1.0558x over previous
import functools

import jax
import jax.numpy as jnp
from jax import lax
from jax.experimental import pallas as pl
from jax.experimental.pallas import tpu as pltpu
from jax.experimental.pallas import tpu_sc as plsc

_NT = 32
_CHUNK = 128
_MCH = 256
_WSEG = 128
_CROWS = 160
_NEG_INF = float("-inf")


def _tc_uv(x, posp, w1, w2p, b2):
    n, d = x.shape
    blk = 2000

    def body(x_ref, p_ref, w1_ref, w2_ref, b_ref, u_ref, v_ref):
        pw = jnp.dot(p_ref[...], w2_ref[...], preferred_element_type=jnp.float32)
        u_ref[...] = jnp.dot(x_ref[...], w1_ref[...],
                             preferred_element_type=jnp.float32) + pw
        v_ref[...] = b_ref[...] - pw

    return pl.pallas_call(
        body,
        grid=(n // blk,),
        in_specs=[
            pl.BlockSpec((blk, d), lambda i: (i, 0)),
            pl.BlockSpec((blk, 128), lambda i: (i, 0)),
            pl.BlockSpec((d, 128), lambda i: (0, 0)),
            pl.BlockSpec((128, 128), lambda i: (0, 0)),
            pl.BlockSpec((1, 128), lambda i: (0, 0)),
        ],
        out_specs=[pl.BlockSpec((blk, 128), lambda i: (i, 0)),
                   pl.BlockSpec((blk, 128), lambda i: (i, 0))],
        out_shape=[jax.ShapeDtypeStruct((n, 128), jnp.float32)] * 2,
    )(x, posp, w1, w2p, b2)


def _tc_epilogue(raw, c):
    rows = raw.shape[0]
    blk = 512

    def body(r_ref, c_ref, o_ref):
        o_ref[...] = jnp.maximum(r_ref[...] + c_ref[...], 0.0)

    return pl.pallas_call(
        body,
        grid=(rows // blk,),
        in_specs=[pl.BlockSpec((blk, 128), lambda i: (i, 0)),
                  pl.BlockSpec((blk, 128), lambda i: (i, 0))],
        out_specs=pl.BlockSpec((blk, 128), lambda i: (i, 0)),
        out_shape=jax.ShapeDtypeStruct((rows, 128), jnp.float32),
    )(raw, c)


def _make_meta(dst_all, etot, nseg2):
    t = jnp.arange(_NT)
    eb = (t * etot) // _NT
    lo = dst_all[eb].at[0].set(0).astype(jnp.int32)
    hi = jnp.concatenate([lo[1:], jnp.array([nseg2], jnp.int32)])
    e_lo = jnp.searchsorted(dst_all, lo, side='left').astype(jnp.int32)
    e_hi = jnp.searchsorted(dst_all, hi, side='left').astype(jnp.int32)
    c0 = (e_lo // _MCH) * _MCH
    nch = jnp.where(e_hi > e_lo, (e_hi - c0 + _MCH - 1) // _MCH, 0)
    z = jnp.zeros_like(c0)
    return jnp.stack([c0, nch, e_lo, e_hi, lo, hi, z, z, z, z, z, z, z, z,
                      z, z], axis=1).astype(jnp.int32)


def _sc_segmax(u, v, idx2, src_all, dst_all, meta, nseg2, nrows):
    mesh = plsc.VectorSubcoreMesh(core_axis_name="c", subcore_axis_name="s",
                                  num_cores=2, num_subcores=16)

    @functools.partial(
        pl.kernel,
        out_type=[jax.ShapeDtypeStruct((nrows, 128), jnp.float32),
                  jax.ShapeDtypeStruct((nrows, 128), jnp.float32)],
        mesh=mesh,
        scratch_types=[
            pltpu.VMEM((16,), jnp.int32),
            pltpu.VMEM((_MCH,), jnp.int32),
            pltpu.VMEM((_MCH,), jnp.int32),
            pltpu.VMEM((_MCH + 16,), jnp.int32),
            pltpu.VMEM((_MCH + 16,), jnp.int32),
            pltpu.VMEM((_MCH, 128), jnp.float32),
            pltpu.VMEM((_MCH, 128), jnp.float32),
            pltpu.VMEM((_WSEG, 128), jnp.float32),
            pltpu.VMEM((_WSEG,), jnp.int32),
            pltpu.VMEM((_CROWS // 2,), jnp.int32),
            pltpu.VMEM((_CROWS // 2,), jnp.int32),
            pltpu.VMEM((_CROWS // 2, 128), jnp.float32),
            pltpu.SemaphoreType.DMA,
            pltpu.SemaphoreType.DMA,
            pltpu.SemaphoreType.DMA,
            pltpu.SemaphoreType.DMA,
        ],
    )
    def k(u_hbm, v_hbm, idx_hbm, src_hbm, dst_hbm, meta_hbm,
          raw_hbm, c_hbm,
          meta_v, src_a, src_b, dst_a, dst_b, rows_a, rows_b,
          win_v, sidx_v, cgi_v, civ_v, crow_v, gsem, isem_a, isem_b, ssem):
        wid = lax.axis_index("s") * 2 + lax.axis_index("c")
        iota = lax.iota(jnp.int32, 16)
        ninf_vec = jnp.full((16,), _NEG_INF, jnp.float32)
        half = _CROWS // 2

        cbase = pl.multiple_of(wid * _CROWS, 8)
        for h in range(2):
            for j in range(half // 16):
                g = cbase + h * half + j * 16 + iota
                cgi_v[pl.ds(j * 16, 16)] = jnp.minimum(g, nseg2 - 1)
            pltpu.async_copy(idx_hbm.at[cgi_v], civ_v, gsem).wait()
            pltpu.async_copy(v_hbm.at[civ_v], crow_v, gsem).wait()
            pltpu.sync_copy(crow_v,
                            c_hbm.at[pl.ds(cbase + h * half, half)])

        pltpu.sync_copy(meta_hbm.at[wid], meta_v)
        mvec = meta_v[...]
        c0 = mvec[0]
        nch = mvec[1]
        e_lo = mvec[2]
        e_hi = mvec[3]
        s_lo = mvec[4]
        s_hi = mvec[5]
        last = jnp.maximum(nch - 1, 0)

        def cstart_of(ci):
            return pl.multiple_of(c0 + jnp.minimum(ci, last) * _MCH, _CHUNK)

        def issue_sd(ci, src_v, dst_v, isem):
            cs = cstart_of(ci)
            pltpu.async_copy(src_hbm.at[pl.ds(cs, _MCH)], src_v, isem)
            pltpu.async_copy(dst_hbm.at[pl.ds(cs, _MCH)],
                             dst_v.at[pl.ds(0, _MCH)], isem)

        def wait_sd(src_v, dst_v, isem):
            pltpu.make_async_copy(src_hbm.at[pl.ds(0, _MCH)], src_v,
                                  isem).wait()
            pltpu.make_async_copy(dst_hbm.at[pl.ds(0, _MCH)],
                                  dst_v.at[pl.ds(0, _MCH)], isem).wait()

        def issue_g(src_v, rows_v):
            for q in range(_MCH // _CHUNK):
                pltpu.async_copy(
                    u_hbm.at[src_v.at[pl.ds(q * _CHUNK, _CHUNK)]],
                    rows_v.at[pl.ds(q * _CHUNK, _CHUNK)], gsem)

        def wait_g(src_v, rows_v):
            for q in range(_MCH // _CHUNK):
                pltpu.make_async_copy(
                    u_hbm.at[src_v.at[pl.ds(q * _CHUNK, _CHUNK)]],
                    rows_v.at[pl.ds(q * _CHUNK, _CHUNK)], gsem).wait()

        def _reinit_win():
            @pl.loop(0, _WSEG)
            def _ib(r):
                wr = win_v.at[r]
                for j in range(8):
                    wr[pl.ds(j * 16, 16)] = ninf_vec

        _reinit_win()

        def flush_window(wbase):
            for j in range(8):
                g = wbase + j * 16 + iota
                sidx_v[pl.ds(j * 16, 16)] = jnp.where(g < s_hi, g, nseg2)
            pltpu.async_copy(win_v, raw_hbm.at[sidx_v], ssem).wait()
            _reinit_win()
            return wbase + _WSEG

        def scan_chunk(ci, carry, src_v, dst_v, rows_v):
            cur, wbase = carry[0], carry[1]
            acc = list(carry[2:])
            cstart = cstart_of(ci)
            lo = jnp.maximum(e_lo, cstart)
            hi = jnp.where(ci < nch, jnp.minimum(e_hi, cstart + _MCH), lo)

            @pl.loop(lo, hi, init_carry=(cur, wbase, *acc))
            def edge_loop(e, ec):
                cur, wbase = ec[0], ec[1]
                acc = list(ec[2:])
                o = e - cstart
                d = dst_v[pl.ds(o, 16)][0]
                ro = rows_v.at[o]
                rv = [ro[pl.ds(j * 16, 16)] for j in range(8)]
                same = d == cur

                @pl.when(jnp.logical_not(same))
                def _():
                    wr = win_v.at[cur - wbase]
                    for jj in range(8):
                        wr[pl.ds(jj * 16, 16)] = acc[jj]

                nadv = (d - wbase) // _WSEG

                @pl.loop(0, nadv, init_carry=wbase)
                def wadv(i, wb):
                    return flush_window(wb)

                new_acc = [
                    jnp.maximum(jnp.where(same, acc[j], ninf_vec), rv[j])
                    for j in range(8)
                ]
                return (d, wadv, *new_acc)

            return edge_loop

        npairs = (nch + 1) // 2

        issue_sd(0, src_a, dst_a, isem_a)
        wait_sd(src_a, dst_a, isem_a)
        issue_g(src_a, rows_a)
        issue_sd(1, src_b, dst_b, isem_b)

        init = (s_lo, s_lo) + tuple(ninf_vec for _ in range(8))

        @pl.loop(0, npairs, init_carry=init)
        def pair_loop(t, carry):
            ca = 2 * t
            wait_g(src_a, rows_a)
            wait_sd(src_b, dst_b, isem_b)
            issue_g(src_b, rows_b)
            carry = scan_chunk(ca, carry, src_a, dst_a, rows_a)
            issue_sd(ca + 2, src_a, dst_a, isem_a)
            wait_g(src_b, rows_b)
            wait_sd(src_a, dst_a, isem_a)
            issue_g(src_a, rows_a)
            carry = scan_chunk(ca + 1, carry, src_b, dst_b, rows_b)
            issue_sd(ca + 3, src_b, dst_b, isem_b)
            return carry

        wait_g(src_a, rows_a)
        wait_sd(src_b, dst_b, isem_b)

        cur, wbase = pair_loop[0], pair_loop[1]
        acc = list(pair_loop[2:])

        @pl.when(e_hi > e_lo)
        def _():
            wr = win_v.at[cur - wbase]
            for j in range(8):
                wr[pl.ds(j * 16, 16)] = acc[j]

        nrem = (s_hi - wbase + _WSEG - 1) // _WSEG

        @pl.loop(0, nrem, init_carry=wbase)
        def wfin(i, wb):
            return flush_window(wb)

    return k(u, v, idx2, src_all, dst_all, meta)


@jax.jit
def kernel(x, pos, idx, src0, dst0, src1, dst1, W, b):
    n, d = x.shape
    nseg = idx.shape[0]
    posp = jnp.zeros((n, 128), jnp.float32).at[:, :3].set(pos)
    w1 = W[:d]
    w2p = jnp.zeros((128, 128), jnp.float32).at[:3].set(W[d:])
    b2 = b.reshape(1, 128)
    u, v = _tc_uv(x, posp, w1, w2p, b2)

    idx2 = jnp.concatenate([idx, idx]).astype(jnp.int32)
    src_all = jnp.concatenate([src0, src1]).astype(jnp.int32)
    dst_all = jnp.concatenate(
        [dst0.astype(jnp.int32), dst1.astype(jnp.int32) + nseg])
    nseg2 = 2 * nseg
    nrows = _NT * _CROWS
    meta = _make_meta(dst_all, src_all.shape[0], nseg2)

    raw, c = _sc_segmax(u, v, idx2, src_all, dst_all, meta, nseg2, nrows)
    out = _tc_epilogue(raw, c)
    return jnp.concatenate([out[:nseg], out[nseg:nseg2]], axis=-1)
